# baseline (speedup 1.0000x reference)
import jax
import jax.numpy as jnp
from jax import lax
from jax.experimental import pallas as pl
from jax.experimental.pallas import tpu as pltpu

_F32 = jnp.float32
_BF16 = jnp.bfloat16

_NG = 64
_NN = 2560
_HID = 64
_W = 4 * _HID
_CR = 10
_CRP = 16
_LC = 16
_GH = 128
_NPROG = 2
_HB = _NG // _NPROG
_HN = _NN // _NPROG
_ODIM = 8

_S_GW = (0, 256, 512)
_S_WIE, _S_WHH0, _S_WIH1 = 768, 1024, 1280
_S_WHH1, _S_FW1, _S_FW2 = 1536, 1792, 2048
_S_WGM, _S_MSEL, _S_BIAS = 2304, 2320, 2336

_R_WIE, _R_WHH0, _R_WIH1 = 0, 128, 384
_R_WHH1, _R_FW1, _R_FW2 = 640, 896, 1152
_R_WGM, _R_MSEL, _R_BIAS = 1408, 1424, 1440
_R_ROWS = 1448


def _body(a_ref, x_ref, p_ref, gm_ref, gw_ref, ws_ref, o_ref, pre_s, hseq_s):
    a = a_ref[0]

    h = x_ref[0]
    y = None
    for i in range(3):
        t = jnp.dot(a, h, preferred_element_type=_F32)
        y = jnp.dot(t.astype(_BF16), gw_ref[i * _GH:(i + 1) * _GH, :],
                    preferred_element_type=_F32)
        y = y + ws_ref[_R_BIAS + i:_R_BIAS + i + 1, :_GH]
        if i < 2:
            y = jnp.maximum(y, 0.0)
        h = y.astype(_BF16)
    emb = jnp.dot(p_ref[0], y, preferred_element_type=_F32)

    emb_g = (jnp.dot(emb, ws_ref[_R_WIE:_R_WIE + _GH, :],
                     preferred_element_type=_F32)
             + ws_ref[_R_BIAS + 3:_R_BIAS + 4, :])

    gm = gm_ref[0]
    pre = jnp.dot(gm, ws_ref[_R_WGM:_R_WGM + _CRP, :],
                  preferred_element_type=_F32)
    for t in range(_LC):
        pre_s[t * _HB:(t + 1) * _HB, :] = pre[t * _HB:(t + 1) * _HB, :] + emb_g

    lane = lax.broadcasted_iota(jnp.int32, (_HB, _W), 1)
    g_sel = (lane >= 2 * _HID) & (lane < 3 * _HID)
    whh0 = ws_ref[_R_WHH0:_R_WHH0 + _W, :]
    wih1 = ws_ref[_R_WIH1:_R_WIH1 + _W, :]
    whh1 = ws_ref[_R_WHH1:_R_WHH1 + _W, :]
    b1 = ws_ref[_R_BIAS + 4:_R_BIAS + 5, :]

    def cell(gates, c_old):
        s = jax.nn.sigmoid(jnp.where(g_sel, gates + gates, gates))
        act = jnp.where(g_sel, s + s - 1.0, s)
        f_al = pltpu.roll(act, 3 * _HID, 1)
        g_al = pltpu.roll(act, 2 * _HID, 1)
        o_al = pltpu.roll(act, _HID, 1)
        c_new = f_al * c_old + act * g_al
        h_new = o_al * jnp.tanh(c_new)
        return h_new, c_new

    zeros = jnp.zeros((_HB, _W), _F32)
    h0, c0, h1, c1 = zeros, zeros, zeros, zeros
    for t in range(_LC):
        g0 = (pre_s[t * _HB:(t + 1) * _HB, :]
              + jnp.dot(h0, whh0, preferred_element_type=_F32))
        h0, c0 = cell(g0, c0)
        g1 = (jnp.dot(h0, wih1, preferred_element_type=_F32)
              + jnp.dot(h1, whh1, preferred_element_type=_F32) + b1)
        h1, c1 = cell(g1, c1)
        hseq_s[t * _HB:(t + 1) * _HB, :] = h1

    mask = jnp.dot(gm, ws_ref[_R_MSEL:_R_MSEL + _CRP, :],
                   preferred_element_type=_F32)
    hm = hseq_s[...] * mask
    yh = jnp.maximum(jnp.dot(hm, ws_ref[_R_FW1:_R_FW1 + _W, :],
                             preferred_element_type=_F32)
                     + ws_ref[_R_BIAS + 5:_R_BIAS + 6, :], 0.0)
    o_ref[0] = (jnp.dot(yh, ws_ref[_R_FW2:_R_FW2 + _W, :],
                        preferred_element_type=_F32)
                + ws_ref[_R_BIAS + 6:_R_BIAS + 7, :])


def kernel(wslab, x_pad, adj, pool_pad, ground_motion, time_steps):
    f32 = _F32
    adj_h = jnp.stack([adj[:_HN, :_HN], adj[_HN:, _HN:]]).astype(_BF16)
    x_h = jnp.stack([x_pad[:_HN, :_GH], x_pad[_HN:, :_GH]]).astype(_BF16)
    pool_h = jnp.stack([pool_pad[:_HB, :_HN], pool_pad[_HB:, _HN:]])

    gw = jnp.concatenate([wslab[o:o + _GH, :_GH] for o in _S_GW]).astype(_BF16)
    ws = jnp.concatenate([
        wslab[_S_WIE:_S_WIE + _GH, :],
        wslab[_S_WHH0:_S_WHH0 + _W, :],
        wslab[_S_WIH1:_S_WIH1 + _W, :],
        wslab[_S_WHH1:_S_WHH1 + _W, :],
        wslab[_S_FW1:_S_FW1 + _W, :],
        wslab[_S_FW2:_S_FW2 + _W, :],
        wslab[_S_WGM:_S_WGM + _CRP, :],
        wslab[_S_MSEL:_S_MSEL + _CRP, :],
        wslab[_S_BIAS:_S_BIAS + 8, :],
    ])

    gm = ground_motion.reshape(_NG, _LC, _CR).astype(f32)
    comp_len = jnp.floor(time_steps.astype(f32) / _CR)
    mask_bt = (jnp.arange(_LC, dtype=f32)[None, :] < comp_len[:, None]).astype(f32)
    extra = jnp.zeros((_NG, _LC, _CRP - _CR), f32).at[:, :, 0].set(mask_bt)
    gmt = jnp.transpose(jnp.concatenate([gm, extra], axis=2), (1, 0, 2))
    gm_h = jnp.stack([gmt[:, :_HB].reshape(_LC * _HB, _CRP),
                      gmt[:, _HB:].reshape(_LC * _HB, _CRP)])

    out = pl.pallas_call(
        _body,
        out_shape=jax.ShapeDtypeStruct((_NPROG, _LC * _HB, _W), f32),
        grid=(_NPROG,),
        in_specs=[
            pl.BlockSpec((1, _HN, _HN), lambda i: (i, 0, 0)),
            pl.BlockSpec((1, _HN, _GH), lambda i: (i, 0, 0)),
            pl.BlockSpec((1, _HB, _HN), lambda i: (i, 0, 0)),
            pl.BlockSpec((1, _LC * _HB, _CRP), lambda i: (i, 0, 0)),
            pl.BlockSpec((3 * _GH, _GH), lambda i: (0, 0)),
            pl.BlockSpec((_R_ROWS, _W), lambda i: (0, 0)),
        ],
        out_specs=pl.BlockSpec((1, _LC * _HB, _W), lambda i: (i, 0, 0)),
        scratch_shapes=[pltpu.VMEM((_LC * _HB, _W), f32),
                        pltpu.VMEM((_LC * _HB, _W), f32)],
        compiler_params=pltpu.CompilerParams(
            dimension_semantics=("parallel",)),
    )(adj_h, x_h, pool_h, gm_h, gw, ws)

    out = out.reshape(_NPROG, _LC, _HB, _W)
    out = jnp.transpose(out, (0, 2, 1, 3)).reshape(_NG, _LC, _W)
    return out[:, :, :_ODIM]


# speedup vs baseline: 1.4838x; 1.4838x over previous
import jax
import jax.numpy as jnp
from jax import lax
from jax.experimental import pallas as pl
from jax.experimental.pallas import tpu as pltpu

_F32 = jnp.float32
_BF16 = jnp.bfloat16

_NG = 64
_NN = 2560
_HID = 64
_W = 4 * _HID
_CR = 10
_CRP = 16
_LC = 16
_GH = 128
_NPROG = 2
_HB = _NG // _NPROG
_HN = _NN // _NPROG
_ODIM = 8

_S_GW = (0, 256, 512)
_S_WIE, _S_WHH0, _S_WIH1 = 768, 1024, 1280
_S_WHH1, _S_FW1, _S_FW2 = 1536, 1792, 2048
_S_WGM, _S_MSEL, _S_BIAS = 2304, 2320, 2336


def _body(a_ref, x_ref, p_ref, gm_ref, w_ref, o_ref, pre_s, hseq_s):
    def brow(k, lanes=_W):
        r = _S_BIAS + k
        return w_ref[r:r + 1, :lanes]

    a = a_ref[...].astype(_BF16)

    h = x_ref[...].astype(_BF16)
    y = None
    for i in range(3):
        t = jnp.dot(a, h, preferred_element_type=_F32)
        gw = w_ref[_S_GW[i]:_S_GW[i] + _GH, :_GH].astype(_BF16)
        y = jnp.dot(t.astype(_BF16), gw, preferred_element_type=_F32)
        y = y + brow(i, _GH)
        if i < 2:
            y = jnp.maximum(y, 0.0)
        h = y.astype(_BF16)
    emb = jnp.dot(p_ref[...], y, preferred_element_type=_F32)

    emb_g = (jnp.dot(emb, w_ref[_S_WIE:_S_WIE + _GH, :],
                     preferred_element_type=_F32) + brow(3))

    gm = gm_ref[0]
    pre = jnp.dot(gm, w_ref[_S_WGM:_S_WGM + _CRP, :],
                  preferred_element_type=_F32)
    for t in range(_LC):
        pre_s[t * _HB:(t + 1) * _HB, :] = pre[t * _HB:(t + 1) * _HB, :] + emb_g

    lane = lax.broadcasted_iota(jnp.int32, (_HB, _W), 1)
    g_sel = (lane >= 2 * _HID) & (lane < 3 * _HID)
    whh0 = w_ref[_S_WHH0:_S_WHH0 + _W, :]
    wih1 = w_ref[_S_WIH1:_S_WIH1 + _W, :]
    whh1 = w_ref[_S_WHH1:_S_WHH1 + _W, :]
    b1 = brow(4)

    def cell(gates, c_old):
        s = jax.nn.sigmoid(jnp.where(g_sel, gates + gates, gates))
        act = jnp.where(g_sel, s + s - 1.0, s)
        f_al = pltpu.roll(act, 3 * _HID, 1)
        g_al = pltpu.roll(act, 2 * _HID, 1)
        o_al = pltpu.roll(act, _HID, 1)
        c_new = f_al * c_old + act * g_al
        h_new = o_al * jnp.tanh(c_new)
        return h_new, c_new

    zeros = jnp.zeros((_HB, _W), _F32)
    h0, c0, h1, c1 = zeros, zeros, zeros, zeros
    for t in range(_LC):
        g0 = (pre_s[t * _HB:(t + 1) * _HB, :]
              + jnp.dot(h0, whh0, preferred_element_type=_F32))
        h0, c0 = cell(g0, c0)
        g1 = (jnp.dot(h0, wih1, preferred_element_type=_F32)
              + jnp.dot(h1, whh1, preferred_element_type=_F32) + b1)
        h1, c1 = cell(g1, c1)
        hseq_s[t * _HB:(t + 1) * _HB, :] = h1

    mask = jnp.dot(gm, w_ref[_S_MSEL:_S_MSEL + _CRP, :],
                   preferred_element_type=_F32)
    hm = hseq_s[...] * mask
    yh = jnp.maximum(jnp.dot(hm, w_ref[_S_FW1:_S_FW1 + _W, :],
                             preferred_element_type=_F32) + brow(5), 0.0)
    o_ref[0] = (jnp.dot(yh, w_ref[_S_FW2:_S_FW2 + _W, :],
                        preferred_element_type=_F32) + brow(6))


def kernel(wslab, x_pad, adj, pool_pad, ground_motion, time_steps):
    f32 = _F32
    gm = ground_motion.reshape(_NG, _LC, _CR).astype(f32)
    comp_len = jnp.floor(time_steps.astype(f32) / _CR)
    mask_bt = (jnp.arange(_LC, dtype=f32)[None, :] < comp_len[:, None]).astype(f32)
    extra = jnp.zeros((_NG, _LC, _CRP - _CR), f32).at[:, :, 0].set(mask_bt)
    gmt = jnp.transpose(jnp.concatenate([gm, extra], axis=2), (1, 0, 2))
    gm_h = jnp.stack([gmt[:, :_HB].reshape(_LC * _HB, _CRP),
                      gmt[:, _HB:].reshape(_LC * _HB, _CRP)])

    out = pl.pallas_call(
        _body,
        out_shape=jax.ShapeDtypeStruct((_NPROG, _LC * _HB, _W), f32),
        grid=(_NPROG,),
        in_specs=[
            pl.BlockSpec((_HN, _HN), lambda i: (i, i)),
            pl.BlockSpec((_HN, _GH), lambda i: (i, 0)),
            pl.BlockSpec((_HB, _HN), lambda i: (i, i)),
            pl.BlockSpec((1, _LC * _HB, _CRP), lambda i: (i, 0, 0)),
            pl.BlockSpec(wslab.shape, lambda i: (0, 0)),
        ],
        out_specs=pl.BlockSpec((1, _LC * _HB, _W), lambda i: (i, 0, 0)),
        scratch_shapes=[pltpu.VMEM((_LC * _HB, _W), f32),
                        pltpu.VMEM((_LC * _HB, _W), f32)],
        compiler_params=pltpu.CompilerParams(
            dimension_semantics=("parallel",)),
    )(adj, x_pad, pool_pad, gm_h, wslab)

    out = out.reshape(_NPROG, _LC, _HB, _W)
    out = jnp.transpose(out, (0, 2, 1, 3)).reshape(_NG, _LC, _W)
    return out[:, :, :_ODIM]
